# Initial kernel scaffold; baseline (speedup 1.0000x reference)
import jax
import jax.numpy as jnp
from jax.experimental import pallas as pl


def kernel(x, W):
    raise NotImplementedError("write your pallas kernel here")


# speedup vs baseline: 1.0589x; 1.0589x over previous
import jax
import jax.numpy as jnp
from jax import lax
from jax.experimental import pallas as pl
from jax.experimental.pallas import tpu as pltpu
from jax.experimental.pallas import tpu_sc as plsc

_NUM_Q = 8
_K = 8192
_D = 32
_CC = 0.25
_N_TOK = 8192
_TILE = 256
_GRID = _N_TOK // _TILE

_NC = 2
_NS = 16
_NW = _NC * _NS
_TPW = _N_TOK // _NW
_ICHUNK = 128
_LANES = 16
_WPAD = 128


def _tc_body(rho_ref, w2t_ref, wn_ref, rn_ref, idx_ref, loss_ref):
    rho = rho_ref[...]
    m2 = lax.dot_general(rho, w2t_ref[...],
                         (((1,), (0,)), ((), ())),
                         preferred_element_type=jnp.float32)
    dist = (rn_ref[0] + wn_ref[...]) - m2
    rowmin = jnp.min(dist, axis=1, keepdims=True)
    iota = lax.broadcasted_iota(jnp.int32, (_TILE, _K), 1)
    cand = jnp.where(dist == rowmin, iota, jnp.int32(_K))
    idx_ref[0] = jnp.min(cand, axis=1, keepdims=True)
    loss_ref[0] = jnp.sum(rowmin, keepdims=True)


def _tc_dist_argmin(rho, w2t, wn2, rn3):
    idx3, loss3 = pl.pallas_call(
        _tc_body,
        grid=(_GRID,),
        in_specs=[
            pl.BlockSpec((_TILE, _D), lambda i: (i, 0)),
            pl.BlockSpec((_D, _K), lambda i: (0, 0)),
            pl.BlockSpec((1, _K), lambda i: (0, 0)),
            pl.BlockSpec((1, _TILE, 1), lambda i: (i, 0, 0)),
        ],
        out_specs=[
            pl.BlockSpec((1, _TILE, 1), lambda i: (i, 0, 0)),
            pl.BlockSpec((1, 1, 1), lambda i: (i, 0, 0)),
        ],
        out_shape=[
            jax.ShapeDtypeStruct((_GRID, _TILE, 1), jnp.int32),
            jax.ShapeDtypeStruct((_GRID, 1, 1), jnp.float32),
        ],
    )(rho, w2t, wn2, rn3)
    return idx3.reshape(_N_TOK), loss3


def _sc_worker_id():
    return lax.axis_index("s") * _NC + lax.axis_index("c")


def _sc_gather_sub_body(w_hbm, idx_hbm, rho_hbm, out_hbm, idx_v, rows_v, rho_v):
    wid = _sc_worker_id()
    base = wid * _TPW
    nrow = _TPW // _ICHUNK
    pltpu.sync_copy(idx_hbm.at[pl.ds(wid * nrow, nrow)], idx_v)
    pltpu.sync_copy(rho_hbm.at[pl.ds(base, _TPW)], rho_v)
    for j in range(nrow):
        pltpu.sync_copy(w_hbm.at[idx_v.at[j]],
                        rows_v.at[pl.ds(j * _ICHUNK, _ICHUNK)])

    def sub(t, carry):
        for dd in range(0, _D, _LANES):
            sl = pl.ds(dd, _LANES)
            rho_v[t, sl] = rho_v[t, sl] - rows_v[t, sl]
        return carry

    lax.fori_loop(0, _TPW, sub, 0)
    pltpu.sync_copy(rho_v, out_hbm.at[pl.ds(base, _TPW)])


def _sc_gather_sub_final_body(w_hbm, idx_hbm, rho_hbm, x_hbm, out_hbm,
                              idx_v, rows_v, rho_v, x_v):
    wid = _sc_worker_id()
    base = wid * _TPW
    nrow = _TPW // _ICHUNK
    pltpu.sync_copy(idx_hbm.at[pl.ds(wid * nrow, nrow)], idx_v)
    pltpu.sync_copy(rho_hbm.at[pl.ds(base, _TPW)], rho_v)
    pltpu.sync_copy(x_hbm.at[pl.ds(base, _TPW)], x_v)
    for j in range(nrow):
        pltpu.sync_copy(w_hbm.at[idx_v.at[j]],
                        rows_v.at[pl.ds(j * _ICHUNK, _ICHUNK)])

    def sub(t, carry):
        for dd in range(0, _D, _LANES):
            sl = pl.ds(dd, _LANES)
            r = rho_v[t, sl] - rows_v[t, sl]
            rho_v[t, sl] = x_v[t, sl] - r
        return carry

    lax.fori_loop(0, _TPW, sub, 0)
    pltpu.sync_copy(rho_v, out_hbm.at[pl.ds(base, _TPW)])


import functools


@functools.lru_cache(maxsize=None)
def _sc_kernels():
    mesh = plsc.VectorSubcoreMesh(core_axis_name="c", subcore_axis_name="s")
    mid = pl.kernel(
        _sc_gather_sub_body,
        out_type=jax.ShapeDtypeStruct((_N_TOK, _D), jnp.float32),
        mesh=mesh,
        scratch_types=[
            pltpu.VMEM((_TPW // _ICHUNK, _ICHUNK), jnp.int32),
            pltpu.VMEM((_TPW, _WPAD), jnp.float32),
            pltpu.VMEM((_TPW, _D), jnp.float32),
        ],
    )
    final = pl.kernel(
        _sc_gather_sub_final_body,
        out_type=jax.ShapeDtypeStruct((_N_TOK, _D), jnp.float32),
        mesh=mesh,
        scratch_types=[
            pltpu.VMEM((_TPW // _ICHUNK, _ICHUNK), jnp.int32),
            pltpu.VMEM((_TPW, _WPAD), jnp.float32),
            pltpu.VMEM((_TPW, _D), jnp.float32),
            pltpu.VMEM((_TPW, _D), jnp.float32),
        ],
    )
    return mid, final


def kernel(x, W):
    B, T, D = x.shape
    flat = x.reshape(_N_TOK, D)
    w2t = jnp.swapaxes(2.0 * W, 1, 2)
    wn = jnp.sum(W ** 2, axis=2)
    wpad = jnp.pad(W, ((0, 0), (0, 0), (0, _WPAD - _D)))

    rho = flat
    total_loss = jnp.float32(0.0)
    all_idx = []
    inv_n = jnp.float32(1.0 / (_N_TOK * _D))
    out = None
    for i in range(_NUM_Q):
        rn = jnp.sum(rho ** 2, axis=1)
        idx, loss3 = _tc_dist_argmin(
            rho, w2t[i], wn[i].reshape(1, _K), rn.reshape(_GRID, _TILE, 1))
        idx2 = idx.reshape(_N_TOK // _ICHUNK, _ICHUNK)
        sc_mid, sc_final = _sc_kernels()
        if i + 1 < _NUM_Q:
            rho = sc_mid(wpad[i], idx2, rho)
        else:
            out = sc_final(wpad[i], idx2, rho, flat)
        m = jnp.sum(loss3) * inv_n
        total_loss = total_loss + (m + _CC * m)
        all_idx.append(idx.reshape(B, T))

    output = out.reshape(B, T, D)
    return output, total_loss, jnp.stack(all_idx, axis=0)


# speedup vs baseline: 1.1103x; 1.0485x over previous
import jax
import jax.numpy as jnp
from jax import lax
from jax.experimental import pallas as pl
from jax.experimental.pallas import tpu as pltpu
from jax.experimental.pallas import tpu_sc as plsc

_NUM_Q = 8
_K = 8192
_D = 32
_CC = 0.25
_N_TOK = 8192
_TILE = 512
_GRID = _N_TOK // _TILE

_NC = 2
_NS = 16
_NW = _NC * _NS
_TPW = _N_TOK // _NW
_ICHUNK = 128
_LANES = 16
_WPAD = 128


def _tc_body(rho_ref, w2t_ref, wn_ref, rn_ref, idx_ref, loss_ref):
    rho = rho_ref[...]
    m2 = lax.dot_general(rho, w2t_ref[...],
                         (((1,), (0,)), ((), ())),
                         preferred_element_type=jnp.float32)
    dist = (rn_ref[0] + wn_ref[...]) - m2
    rowmin = jnp.min(dist, axis=1, keepdims=True)
    iota = lax.broadcasted_iota(jnp.int32, (_TILE, _K), 1)
    cand = jnp.where(dist == rowmin, iota, jnp.int32(_K))
    idx_ref[0] = jnp.min(cand, axis=1, keepdims=True)
    loss_ref[0] = jnp.sum(rowmin, keepdims=True)


def _tc_dist_argmin(rho, w2t, wn2, rn3):
    idx3, loss3 = pl.pallas_call(
        _tc_body,
        grid=(_GRID,),
        in_specs=[
            pl.BlockSpec((_TILE, _D), lambda i: (i, 0)),
            pl.BlockSpec((_D, _K), lambda i: (0, 0)),
            pl.BlockSpec((1, _K), lambda i: (0, 0)),
            pl.BlockSpec((1, _TILE, 1), lambda i: (i, 0, 0)),
        ],
        out_specs=[
            pl.BlockSpec((1, _TILE, 1), lambda i: (i, 0, 0)),
            pl.BlockSpec((1, 1, 1), lambda i: (i, 0, 0)),
        ],
        out_shape=[
            jax.ShapeDtypeStruct((_GRID, _TILE, 1), jnp.int32),
            jax.ShapeDtypeStruct((_GRID, 1, 1), jnp.float32),
        ],
    )(rho, w2t, wn2, rn3)
    return idx3.reshape(_N_TOK), loss3


def _sc_worker_id():
    return lax.axis_index("s") * _NC + lax.axis_index("c")


def _sc_gather_sub_body(w_hbm, idx_hbm, rho_hbm, out_hbm, idx_v, rows_v, rho_v):
    wid = _sc_worker_id()
    base = wid * _TPW
    nrow = _TPW // _ICHUNK
    pltpu.sync_copy(idx_hbm.at[pl.ds(wid * nrow, nrow)], idx_v)
    pltpu.sync_copy(rho_hbm.at[pl.ds(base, _TPW)], rho_v)
    for j in range(nrow):
        pltpu.sync_copy(w_hbm.at[idx_v.at[j]],
                        rows_v.at[pl.ds(j * _ICHUNK, _ICHUNK)])

    def sub(t, carry):
        for dd in range(0, _D, _LANES):
            sl = pl.ds(dd, _LANES)
            rho_v[t, sl] = rho_v[t, sl] - rows_v[t, sl]
        return carry

    lax.fori_loop(0, _TPW, sub, 0)
    pltpu.sync_copy(rho_v, out_hbm.at[pl.ds(base, _TPW)])


def _sc_gather_sub_final_body(w_hbm, idx_hbm, rho_hbm, x_hbm, out_hbm,
                              idx_v, rows_v, rho_v, x_v):
    wid = _sc_worker_id()
    base = wid * _TPW
    nrow = _TPW // _ICHUNK
    pltpu.sync_copy(idx_hbm.at[pl.ds(wid * nrow, nrow)], idx_v)
    pltpu.sync_copy(rho_hbm.at[pl.ds(base, _TPW)], rho_v)
    pltpu.sync_copy(x_hbm.at[pl.ds(base, _TPW)], x_v)
    for j in range(nrow):
        pltpu.sync_copy(w_hbm.at[idx_v.at[j]],
                        rows_v.at[pl.ds(j * _ICHUNK, _ICHUNK)])

    def sub(t, carry):
        for dd in range(0, _D, _LANES):
            sl = pl.ds(dd, _LANES)
            r = rho_v[t, sl] - rows_v[t, sl]
            rho_v[t, sl] = x_v[t, sl] - r
        return carry

    lax.fori_loop(0, _TPW, sub, 0)
    pltpu.sync_copy(rho_v, out_hbm.at[pl.ds(base, _TPW)])


import functools


@functools.lru_cache(maxsize=None)
def _sc_kernels():
    mesh = plsc.VectorSubcoreMesh(core_axis_name="c", subcore_axis_name="s")
    mid = pl.kernel(
        _sc_gather_sub_body,
        out_type=jax.ShapeDtypeStruct((_N_TOK, _D), jnp.float32),
        mesh=mesh,
        scratch_types=[
            pltpu.VMEM((_TPW // _ICHUNK, _ICHUNK), jnp.int32),
            pltpu.VMEM((_TPW, _WPAD), jnp.float32),
            pltpu.VMEM((_TPW, _D), jnp.float32),
        ],
    )
    final = pl.kernel(
        _sc_gather_sub_final_body,
        out_type=jax.ShapeDtypeStruct((_N_TOK, _D), jnp.float32),
        mesh=mesh,
        scratch_types=[
            pltpu.VMEM((_TPW // _ICHUNK, _ICHUNK), jnp.int32),
            pltpu.VMEM((_TPW, _WPAD), jnp.float32),
            pltpu.VMEM((_TPW, _D), jnp.float32),
            pltpu.VMEM((_TPW, _D), jnp.float32),
        ],
    )
    return mid, final


def kernel(x, W):
    B, T, D = x.shape
    flat = x.reshape(_N_TOK, D)
    w2t = jnp.swapaxes(2.0 * W, 1, 2)
    wn = jnp.sum(W ** 2, axis=2)
    wpad = jnp.pad(W, ((0, 0), (0, 0), (0, _WPAD - _D)))

    rho = flat
    total_loss = jnp.float32(0.0)
    all_idx = []
    inv_n = jnp.float32(1.0 / (_N_TOK * _D))
    out = None
    for i in range(_NUM_Q):
        rn = jnp.sum(rho ** 2, axis=1)
        idx, loss3 = _tc_dist_argmin(
            rho, w2t[i], wn[i].reshape(1, _K), rn.reshape(_GRID, _TILE, 1))
        idx2 = idx.reshape(_N_TOK // _ICHUNK, _ICHUNK)
        sc_mid, sc_final = _sc_kernels()
        if i + 1 < _NUM_Q:
            rho = sc_mid(wpad[i], idx2, rho)
        else:
            out = sc_final(wpad[i], idx2, rho, flat)
        m = jnp.sum(loss3) * inv_n
        total_loss = total_loss + (m + _CC * m)
        all_idx.append(idx.reshape(B, T))

    output = out.reshape(B, T, D)
    return output, total_loss, jnp.stack(all_idx, axis=0)


# speedup vs baseline: 1.1340x; 1.0213x over previous
import jax
import jax.numpy as jnp
from jax import lax
from jax.experimental import pallas as pl
from jax.experimental.pallas import tpu as pltpu
from jax.experimental.pallas import tpu_sc as plsc

_NUM_Q = 8
_K = 8192
_D = 32
_CC = 0.25
_N_TOK = 8192
_TILE = 1024
_GRID = _N_TOK // _TILE

_NC = 2
_NS = 16
_NW = _NC * _NS
_TPW = _N_TOK // _NW
_ICHUNK = 128
_LANES = 16
_WPAD = 128


def _tc_body(rho_ref, w2t_ref, wn_ref, rn_ref, idx_ref, loss_ref):
    rho = rho_ref[...]
    m2 = lax.dot_general(rho, w2t_ref[...],
                         (((1,), (0,)), ((), ())),
                         preferred_element_type=jnp.float32)
    dist = (rn_ref[0] + wn_ref[...]) - m2
    rowmin = jnp.min(dist, axis=1, keepdims=True)
    iota = lax.broadcasted_iota(jnp.int32, (_TILE, _K), 1)
    cand = jnp.where(dist == rowmin, iota, jnp.int32(_K))
    idx_ref[0] = jnp.min(cand, axis=1, keepdims=True)
    loss_ref[0] = jnp.sum(rowmin, keepdims=True)


def _tc_dist_argmin(rho, w2t, wn2, rn3):
    idx3, loss3 = pl.pallas_call(
        _tc_body,
        grid=(_GRID,),
        in_specs=[
            pl.BlockSpec((_TILE, _D), lambda i: (i, 0)),
            pl.BlockSpec((_D, _K), lambda i: (0, 0)),
            pl.BlockSpec((1, _K), lambda i: (0, 0)),
            pl.BlockSpec((1, _TILE, 1), lambda i: (i, 0, 0)),
        ],
        out_specs=[
            pl.BlockSpec((1, _TILE, 1), lambda i: (i, 0, 0)),
            pl.BlockSpec((1, 1, 1), lambda i: (i, 0, 0)),
        ],
        out_shape=[
            jax.ShapeDtypeStruct((_GRID, _TILE, 1), jnp.int32),
            jax.ShapeDtypeStruct((_GRID, 1, 1), jnp.float32),
        ],
    )(rho, w2t, wn2, rn3)
    return idx3.reshape(_N_TOK), loss3


def _sc_worker_id():
    return lax.axis_index("s") * _NC + lax.axis_index("c")


def _sc_gather_sub_body(w_hbm, idx_hbm, rho_hbm, out_hbm, idx_v, rows_v, rho_v):
    wid = _sc_worker_id()
    base = wid * _TPW
    nrow = _TPW // _ICHUNK
    pltpu.sync_copy(idx_hbm.at[pl.ds(wid * nrow, nrow)], idx_v)
    pltpu.sync_copy(rho_hbm.at[pl.ds(base, _TPW)], rho_v)
    for j in range(nrow):
        pltpu.sync_copy(w_hbm.at[idx_v.at[j]],
                        rows_v.at[pl.ds(j * _ICHUNK, _ICHUNK)])

    def sub(t, carry):
        for dd in range(0, _D, _LANES):
            sl = pl.ds(dd, _LANES)
            rho_v[t, sl] = rho_v[t, sl] - rows_v[t, sl]
        return carry

    lax.fori_loop(0, _TPW, sub, 0)
    pltpu.sync_copy(rho_v, out_hbm.at[pl.ds(base, _TPW)])


def _sc_gather_sub_final_body(w_hbm, idx_hbm, rho_hbm, x_hbm, out_hbm,
                              idx_v, rows_v, rho_v, x_v):
    wid = _sc_worker_id()
    base = wid * _TPW
    nrow = _TPW // _ICHUNK
    pltpu.sync_copy(idx_hbm.at[pl.ds(wid * nrow, nrow)], idx_v)
    pltpu.sync_copy(rho_hbm.at[pl.ds(base, _TPW)], rho_v)
    pltpu.sync_copy(x_hbm.at[pl.ds(base, _TPW)], x_v)
    for j in range(nrow):
        pltpu.sync_copy(w_hbm.at[idx_v.at[j]],
                        rows_v.at[pl.ds(j * _ICHUNK, _ICHUNK)])

    def sub(t, carry):
        for dd in range(0, _D, _LANES):
            sl = pl.ds(dd, _LANES)
            r = rho_v[t, sl] - rows_v[t, sl]
            rho_v[t, sl] = x_v[t, sl] - r
        return carry

    lax.fori_loop(0, _TPW, sub, 0)
    pltpu.sync_copy(rho_v, out_hbm.at[pl.ds(base, _TPW)])


import functools


@functools.lru_cache(maxsize=None)
def _sc_kernels():
    mesh = plsc.VectorSubcoreMesh(core_axis_name="c", subcore_axis_name="s")
    mid = pl.kernel(
        _sc_gather_sub_body,
        out_type=jax.ShapeDtypeStruct((_N_TOK, _D), jnp.float32),
        mesh=mesh,
        scratch_types=[
            pltpu.VMEM((_TPW // _ICHUNK, _ICHUNK), jnp.int32),
            pltpu.VMEM((_TPW, _WPAD), jnp.float32),
            pltpu.VMEM((_TPW, _D), jnp.float32),
        ],
    )
    final = pl.kernel(
        _sc_gather_sub_final_body,
        out_type=jax.ShapeDtypeStruct((_N_TOK, _D), jnp.float32),
        mesh=mesh,
        scratch_types=[
            pltpu.VMEM((_TPW // _ICHUNK, _ICHUNK), jnp.int32),
            pltpu.VMEM((_TPW, _WPAD), jnp.float32),
            pltpu.VMEM((_TPW, _D), jnp.float32),
            pltpu.VMEM((_TPW, _D), jnp.float32),
        ],
    )
    return mid, final


def kernel(x, W):
    B, T, D = x.shape
    flat = x.reshape(_N_TOK, D)
    w2t = jnp.swapaxes(2.0 * W, 1, 2)
    wn = jnp.sum(W ** 2, axis=2)
    wpad = jnp.pad(W, ((0, 0), (0, 0), (0, _WPAD - _D)))

    rho = flat
    total_loss = jnp.float32(0.0)
    all_idx = []
    inv_n = jnp.float32(1.0 / (_N_TOK * _D))
    out = None
    for i in range(_NUM_Q):
        rn = jnp.sum(rho ** 2, axis=1)
        idx, loss3 = _tc_dist_argmin(
            rho, w2t[i], wn[i].reshape(1, _K), rn.reshape(_GRID, _TILE, 1))
        idx2 = idx.reshape(_N_TOK // _ICHUNK, _ICHUNK)
        sc_mid, sc_final = _sc_kernels()
        if i + 1 < _NUM_Q:
            rho = sc_mid(wpad[i], idx2, rho)
        else:
            out = sc_final(wpad[i], idx2, rho, flat)
        m = jnp.sum(loss3) * inv_n
        total_loss = total_loss + (m + _CC * m)
        all_idx.append(idx.reshape(B, T))

    output = out.reshape(B, T, D)
    return output, total_loss, jnp.stack(all_idx, axis=0)
